# baseline (device time: 132158 ns/iter reference)
import jax
import jax.numpy as jnp
from jax import lax
from jax.experimental import pallas as pl
from jax.experimental.pallas import tpu as pltpu

N_DEV = 32
B, SQ, DM = 2, 256, 512
H, DH = 4, 64
HALO = 128
G = 32
W = HALO + SQ + HALO + G
BIG = 10 ** 9
SCALE = 0.125
NEG = -1e9
BF = jnp.bfloat16
F32 = jnp.float32


def kernel(x, Wq, K_ext, V_ext, Wo):
    def body(x_ref, wq_ref, k_ref, v_ref, wo_ref, out_ref,
             ws, bc, part, part_rx, ctxb,
             hs, hr, bs, bcr, ps, prx):
        p = lax.axis_index("i")

        def halo_right():
            return pltpu.make_async_remote_copy(
                src_ref=ws.at[:, :, SQ:HALO + SQ],
                dst_ref=ws.at[:, :, 0:HALO],
                send_sem=hs.at[1], recv_sem=hr.at[0],
                device_id=((p + 1) % N_DEV,),
                device_id_type=pl.DeviceIdType.MESH)

        def halo_left():
            return pltpu.make_async_remote_copy(
                src_ref=ws.at[:, :, HALO:2 * HALO],
                dst_ref=ws.at[:, :, HALO + SQ:HALO + SQ + HALO],
                send_sem=hs.at[0], recv_sem=hr.at[1],
                device_id=((p - 1) % N_DEV,),
                device_id_type=pl.DeviceIdType.MESH)

        def bc_send(t):
            return pltpu.make_async_remote_copy(
                src_ref=bc, dst_ref=bc,
                send_sem=bs.at[t - 1], recv_sem=bcr.at[0],
                device_id=(t,), device_id_type=pl.DeviceIdType.MESH)

        def part_send():
            return pltpu.make_async_remote_copy(
                src_ref=part, dst_ref=part_rx.at[pl.ds(p, 1)],
                send_sem=ps.at[0], recv_sem=prx.at[0],
                device_id=(0,), device_id_type=pl.DeviceIdType.MESH)

        ws[0, :, HALO:HALO + SQ] = k_ref[...]
        ws[1, :, HALO:HALO + SQ] = v_ref[...]

        @pl.when(p == 0)
        def _():
            z = jnp.zeros((B, HALO, H, DH), F32)
            ws[0, :, 0:HALO] = z
            ws[1, :, 0:HALO] = z
            zg = jnp.zeros((B, G, H, DH), F32)
            ws[0, :, HALO + SQ + HALO:] = zg
            ws[1, :, HALO + SQ + HALO:] = zg

        @pl.when(p == N_DEV - 1)
        def _():
            z = jnp.zeros((B, HALO, H, DH), F32)
            ws[0, :, HALO + SQ:HALO + SQ + HALO] = z
            ws[1, :, HALO + SQ:HALO + SQ + HALO] = z

        @pl.when(p < N_DEV - 1)
        def _():
            halo_right().start()

        @pl.when(p > 0)
        def _():
            halo_left().start()

        wq = wq_ref[...].astype(BF)
        qs = []
        for b in range(B):
            qb = lax.dot_general(
                x_ref[b].astype(BF), wq, (((1,), (0,)), ((), ())),
                preferred_element_type=F32)
            qs.append(qb.reshape(SQ, H, DH))

        @pl.when(p == 0)
        def _():
            bc[0] = k_ref[:, 0:G]
            bc[1] = v_ref[:, 0:G]
            for b in range(B):
                bc[2, b] = qs[b][0:G]
            for t in range(1, N_DEV):
                bc_send(t).start()

        @pl.when(p != 0)
        def _():
            bc_send(1).wait_recv()
            ws[0, :, HALO + SQ + HALO:] = bc[0]
            ws[1, :, HALO + SQ + HALO:] = bc[1]

        for b in range(B):
            for h in range(H):
                q32 = bc[2, b, :, h, :].astype(BF)
                kb = k_ref[b, :, h, :].astype(BF)
                s = lax.dot_general(
                    q32, kb, (((1,), (1,)), ((), ())),
                    preferred_element_type=F32) * SCALE
                m = jnp.max(s, axis=1, keepdims=True)
                e = jnp.exp(s - m)
                l = jnp.sum(e, axis=1, keepdims=True)
                vb = v_ref[b, :, h, :].astype(BF)
                pc = lax.dot_general(
                    e.astype(BF), vb, (((1,), (0,)), ((), ())),
                    preferred_element_type=F32)
                part[0, b, h, :, 0:DH] = pc
                part[0, b, h, :, DH:DH + 1] = m
                part[0, b, h, :, DH + 1:DH + 2] = l

        @pl.when(p != 0)
        def _():
            part_send().start()

        @pl.when(p == 0)
        def _():
            part_rx[0] = part[0]

        @pl.when(p > 0)
        def _():
            halo_right().wait_recv()

        @pl.when(p < N_DEV - 1)
        def _():
            halo_left().wait_recv()

        qi = lax.broadcasted_iota(jnp.int32, (SQ, W), 0) + SQ * p
        kl = lax.broadcasted_iota(jnp.int32, (1, HALO), 1) + SQ * p - HALO
        kl = jnp.where(p > 0, kl, BIG)
        kc = lax.broadcasted_iota(jnp.int32, (1, SQ), 1) + SQ * p
        kr = lax.broadcasted_iota(jnp.int32, (1, HALO), 1) + SQ * (p + 1)
        kr = jnp.where(p < N_DEV - 1, kr, BIG)
        kg = lax.broadcasted_iota(jnp.int32, (1, G), 1)
        kg = jnp.where(p > 0, kg, BIG)
        kcol = jnp.concatenate([kl, kc, kr, kg], axis=1)
        mask = ((jnp.abs(qi - kcol) <= HALO) | (kcol < G) | (qi < G))

        for b in range(B):
            for h in range(H):
                qbh = qs[b][:, h, :].astype(BF)
                kws = ws[0, b, :, h, :].astype(BF)
                s = lax.dot_general(
                    qbh, kws, (((1,), (1,)), ((), ())),
                    preferred_element_type=F32) * SCALE
                s = jnp.where(mask, s, NEG)
                m = jnp.max(s, axis=1, keepdims=True)
                e = jnp.exp(s - m)
                l = jnp.sum(e, axis=1, keepdims=True)
                wgt = (e / l).astype(BF)
                vws = ws[1, b, :, h, :].astype(BF)
                c = lax.dot_general(
                    wgt, vws, (((1,), (0,)), ((), ())),
                    preferred_element_type=F32)
                ctxb[b, :, h, :] = c

        wo = wo_ref[...].astype(BF)
        for b in range(B):
            cb = ctxb[b].reshape(SQ, H * DH).astype(BF)
            out_ref[b] = lax.dot_general(
                cb, wo, (((1,), (0,)), ((), ())),
                preferred_element_type=F32)

        @pl.when(p == 0)
        def _():
            rd = part_send()
            for _ in range(N_DEV - 1):
                rd.wait_recv()
            M = part_rx[0][:, :, :, DH:DH + 1]
            for j in range(1, N_DEV):
                M = jnp.maximum(M, part_rx[j][:, :, :, DH:DH + 1])
            L = jnp.zeros((B, H, G, 1), F32)
            C = jnp.zeros((B, H, G, DH), F32)
            for j in range(N_DEV):
                pj = part_rx[j]
                a = jnp.exp(pj[:, :, :, DH:DH + 1] - M)
                L = L + a * pj[:, :, :, DH + 1:DH + 2]
                C = C + a * pj[:, :, :, 0:DH]
            C = C / L
            for h in range(H):
                ctxb[:, 0:G, h, :] = C[:, h]
            for b in range(B):
                cg = ctxb[b, 0:G].reshape(G, H * DH).astype(BF)
                out_ref[b, 0:G] = lax.dot_general(
                    cg, wo, (((1,), (0,)), ((), ())),
                    preferred_element_type=F32)

        @pl.when(p > 0)
        def _():
            halo_left().wait_send()

        @pl.when(p < N_DEV - 1)
        def _():
            halo_right().wait_send()

        @pl.when(p == 0)
        def _():
            for t in range(1, N_DEV):
                bc_send(t).wait_send()

        @pl.when(p != 0)
        def _():
            part_send().wait_send()

    return pl.pallas_call(
        body,
        out_shape=jax.ShapeDtypeStruct((B, SQ, DM), F32),
        in_specs=[pl.BlockSpec(memory_space=pltpu.VMEM)] * 5,
        out_specs=pl.BlockSpec(memory_space=pltpu.VMEM),
        scratch_shapes=[
            pltpu.VMEM((2, B, W, H, DH), F32),
            pltpu.VMEM((3, B, G, H, DH), F32),
            pltpu.VMEM((1, B, H, G, DH + 2), F32),
            pltpu.VMEM((N_DEV, B, H, G, DH + 2), F32),
            pltpu.VMEM((B, SQ, H, DH), F32),
            pltpu.SemaphoreType.DMA((2,)),
            pltpu.SemaphoreType.DMA((2,)),
            pltpu.SemaphoreType.DMA((N_DEV - 1,)),
            pltpu.SemaphoreType.DMA((1,)),
            pltpu.SemaphoreType.DMA((1,)),
            pltpu.SemaphoreType.DMA((1,)),
        ],
    )(x, Wq, K_ext, V_ext, Wo)


# device time: 78857 ns/iter; 1.6759x vs baseline; 1.6759x over previous
import jax
import jax.numpy as jnp
from jax import lax
from jax.experimental import pallas as pl
from jax.experimental.pallas import tpu as pltpu

N_DEV = 32
B, SQ, DM = 2, 256, 512
H, DH = 4, 64
HALO = 128
G = 32
W = HALO + SQ + HALO + G
BIG = 10 ** 9
SCALE = 0.125
NEG = -1e9
BF = jnp.bfloat16
F32 = jnp.float32


def kernel(x, Wq, K_ext, V_ext, Wo):
    def body(x_ref, wq_ref, k_ref, v_ref, wo_ref, out_ref,
             ws, bc, part, part_rx, ctxb,
             hs, hr, bs, bcr, ps, prx):
        p = lax.axis_index("i")

        def halo_right():
            return pltpu.make_async_remote_copy(
                src_ref=ws.at[:, :, SQ:HALO + SQ],
                dst_ref=ws.at[:, :, 0:HALO],
                send_sem=hs.at[1], recv_sem=hr.at[0],
                device_id=((p + 1) % N_DEV,),
                device_id_type=pl.DeviceIdType.MESH)

        def halo_left():
            return pltpu.make_async_remote_copy(
                src_ref=ws.at[:, :, HALO:2 * HALO],
                dst_ref=ws.at[:, :, HALO + SQ:HALO + SQ + HALO],
                send_sem=hs.at[0], recv_sem=hr.at[1],
                device_id=((p - 1) % N_DEV,),
                device_id_type=pl.DeviceIdType.MESH)

        def bc_send(t):
            return pltpu.make_async_remote_copy(
                src_ref=bc, dst_ref=bc,
                send_sem=bs.at[t - 1], recv_sem=bcr.at[0],
                device_id=(t,), device_id_type=pl.DeviceIdType.MESH)

        def part_send():
            return pltpu.make_async_remote_copy(
                src_ref=part, dst_ref=part_rx.at[pl.ds(p, 1)],
                send_sem=ps.at[0], recv_sem=prx.at[0],
                device_id=(0,), device_id_type=pl.DeviceIdType.MESH)

        barrier_sem = pltpu.get_barrier_semaphore()

        @pl.when(p == 0)
        def _():
            for t in range(1, N_DEV):
                pl.semaphore_signal(barrier_sem, inc=1, device_id=(t,),
                                    device_id_type=pl.DeviceIdType.MESH)
            pl.semaphore_wait(barrier_sem, N_DEV - 1)

        @pl.when(p != 0)
        def _():
            pl.semaphore_signal(barrier_sem, inc=1, device_id=(0,),
                                device_id_type=pl.DeviceIdType.MESH)

        @pl.when((p > 0) & (p < N_DEV - 1))
        def _():
            pl.semaphore_signal(barrier_sem, inc=1,
                                device_id=((p + 1) % N_DEV,),
                                device_id_type=pl.DeviceIdType.MESH)

        @pl.when(p > 1)
        def _():
            pl.semaphore_signal(barrier_sem, inc=1, device_id=(p - 1,),
                                device_id_type=pl.DeviceIdType.MESH)

        @pl.when((p == 1) | (p == N_DEV - 1))
        def _():
            pl.semaphore_wait(barrier_sem, 2)

        @pl.when((p > 1) & (p < N_DEV - 1))
        def _():
            pl.semaphore_wait(barrier_sem, 3)

        ws[0, :, HALO:HALO + SQ] = k_ref[...].astype(BF)
        ws[1, :, HALO:HALO + SQ] = v_ref[...].astype(BF)

        @pl.when(p == 0)
        def _():
            z = jnp.zeros((B, HALO, H, DH), BF)
            ws[0, :, 0:HALO] = z
            ws[1, :, 0:HALO] = z
            zg = jnp.zeros((B, G, H, DH), BF)
            ws[0, :, HALO + SQ + HALO:] = zg
            ws[1, :, HALO + SQ + HALO:] = zg

        @pl.when(p == N_DEV - 1)
        def _():
            z = jnp.zeros((B, HALO, H, DH), BF)
            ws[0, :, HALO + SQ:HALO + SQ + HALO] = z
            ws[1, :, HALO + SQ:HALO + SQ + HALO] = z

        @pl.when(p < N_DEV - 1)
        def _():
            halo_right().start()

        @pl.when(p > 0)
        def _():
            halo_left().start()

        wq = wq_ref[...].astype(BF)
        qs = []
        for b in range(B):
            qb = lax.dot_general(
                x_ref[b].astype(BF), wq, (((1,), (0,)), ((), ())),
                preferred_element_type=F32)
            qs.append(qb.reshape(SQ, H, DH))

        @pl.when(p == 0)
        def _():
            bc[0] = k_ref[:, 0:G].astype(BF)
            bc[1] = v_ref[:, 0:G].astype(BF)
            for b in range(B):
                bc[2, b] = qs[b][0:G].astype(BF)
            for t in range(1, N_DEV):
                bc_send(t).start()

        @pl.when(p != 0)
        def _():
            bc_send(1).wait_recv()
            ws[0, :, HALO + SQ + HALO:] = bc[0]
            ws[1, :, HALO + SQ + HALO:] = bc[1]

        for b in range(B):
            for h in range(H):
                q32 = bc[2, b, :, h, :]
                kb = k_ref[b, :, h, :].astype(BF)
                s = lax.dot_general(
                    q32, kb, (((1,), (1,)), ((), ())),
                    preferred_element_type=F32) * SCALE
                m = jnp.max(s, axis=1, keepdims=True)
                e = jnp.exp(s - m)
                l = jnp.sum(e, axis=1, keepdims=True)
                vb = v_ref[b, :, h, :].astype(BF)
                pc = lax.dot_general(
                    e.astype(BF), vb, (((1,), (0,)), ((), ())),
                    preferred_element_type=F32)
                part[0, b, h, :, 0:DH] = pc.astype(BF)
                part[0, b, h, :, DH:DH + 1] = m.astype(BF)
                part[0, b, h, :, DH + 1:DH + 2] = l.astype(BF)

        @pl.when(p != 0)
        def _():
            part_send().start()

        @pl.when(p == 0)
        def _():
            part_rx[0] = part[0]

        @pl.when(p > 0)
        def _():
            halo_right().wait_recv()

        @pl.when(p < N_DEV - 1)
        def _():
            halo_left().wait_recv()

        qi = lax.broadcasted_iota(jnp.int32, (SQ, W), 0) + SQ * p
        kl = lax.broadcasted_iota(jnp.int32, (1, HALO), 1) + SQ * p - HALO
        kl = jnp.where(p > 0, kl, BIG)
        kc = lax.broadcasted_iota(jnp.int32, (1, SQ), 1) + SQ * p
        kr = lax.broadcasted_iota(jnp.int32, (1, HALO), 1) + SQ * (p + 1)
        kr = jnp.where(p < N_DEV - 1, kr, BIG)
        kg = lax.broadcasted_iota(jnp.int32, (1, G), 1)
        kg = jnp.where(p > 0, kg, BIG)
        kcol = jnp.concatenate([kl, kc, kr, kg], axis=1)
        mask = ((jnp.abs(qi - kcol) <= HALO) | (kcol < G) | (qi < G))

        for b in range(B):
            for h in range(H):
                qbh = qs[b][:, h, :].astype(BF)
                kws = ws[0, b, :, h, :]
                s = lax.dot_general(
                    qbh, kws, (((1,), (1,)), ((), ())),
                    preferred_element_type=F32) * SCALE
                s = jnp.where(mask, s, NEG)
                m = jnp.max(s, axis=1, keepdims=True)
                e = jnp.exp(s - m)
                l = jnp.sum(e, axis=1, keepdims=True)
                wgt = (e / l).astype(BF)
                vws = ws[1, b, :, h, :]
                c = lax.dot_general(
                    wgt, vws, (((1,), (0,)), ((), ())),
                    preferred_element_type=F32)
                ctxb[b, :, h, :] = c

        wo = wo_ref[...].astype(BF)
        for b in range(B):
            cb = ctxb[b].reshape(SQ, H * DH).astype(BF)
            out_ref[b] = lax.dot_general(
                cb, wo, (((1,), (0,)), ((), ())),
                preferred_element_type=F32)

        @pl.when(p == 0)
        def _():
            rd = part_send()
            for _ in range(N_DEV - 1):
                rd.wait_recv()
            M = part_rx[0][:, :, :, DH:DH + 1].astype(F32)
            for j in range(1, N_DEV):
                M = jnp.maximum(M, part_rx[j][:, :, :, DH:DH + 1].astype(F32))
            L = jnp.zeros((B, H, G, 1), F32)
            C = jnp.zeros((B, H, G, DH), F32)
            for j in range(N_DEV):
                pj = part_rx[j].astype(F32)
                a = jnp.exp(pj[:, :, :, DH:DH + 1] - M)
                L = L + a * pj[:, :, :, DH + 1:DH + 2]
                C = C + a * pj[:, :, :, 0:DH]
            C = C / L
            for h in range(H):
                ctxb[:, 0:G, h, :] = C[:, h].astype(F32)
            for b in range(B):
                cg = ctxb[b, 0:G].reshape(G, H * DH).astype(BF)
                out_ref[b, 0:G] = lax.dot_general(
                    cg, wo, (((1,), (0,)), ((), ())),
                    preferred_element_type=F32)

        @pl.when(p > 0)
        def _():
            halo_left().wait_send()

        @pl.when(p < N_DEV - 1)
        def _():
            halo_right().wait_send()

        @pl.when(p == 0)
        def _():
            for t in range(1, N_DEV):
                bc_send(t).wait_send()

        @pl.when(p != 0)
        def _():
            part_send().wait_send()

    return pl.pallas_call(
        body,
        out_shape=jax.ShapeDtypeStruct((B, SQ, DM), F32),
        in_specs=[pl.BlockSpec(memory_space=pltpu.VMEM)] * 5,
        out_specs=pl.BlockSpec(memory_space=pltpu.VMEM),
        scratch_shapes=[
            pltpu.VMEM((2, B, W, H, DH), BF),
            pltpu.VMEM((3, B, G, H, DH), BF),
            pltpu.VMEM((1, B, H, G, DH + 2), BF),
            pltpu.VMEM((N_DEV, B, H, G, DH + 2), BF),
            pltpu.VMEM((B, SQ, H, DH), F32),
            pltpu.SemaphoreType.DMA((2,)),
            pltpu.SemaphoreType.DMA((2,)),
            pltpu.SemaphoreType.DMA((N_DEV - 1,)),
            pltpu.SemaphoreType.DMA((1,)),
            pltpu.SemaphoreType.DMA((1,)),
            pltpu.SemaphoreType.DMA((1,)),
        ],
        compiler_params=pltpu.CompilerParams(collective_id=0),
    )(x, Wq, K_ext, V_ext, Wo)
